# baseline (speedup 1.0000x reference)
import functools

import jax
import jax.numpy as jnp
from jax import lax
from jax.experimental import pallas as pl
from jax.experimental.pallas import tpu as pltpu
from jax.experimental.pallas import tpu_sc as plsc

NCORES = 2
NSUB = 16
NW = NCORES * NSUB
CHUNK = 128
SG = 8


@functools.lru_cache(maxsize=None)
def _make_sc_scatter(dt, npad, nch, gather):
    rows_per_tile = npad // NSUB
    mesh = plsc.VectorSubcoreMesh(core_axis_name="c", subcore_axis_name="s",
                                  num_cores=NCORES)
    assert nch % SG == 0
    nsg = nch // SG

    @functools.partial(
        pl.kernel,
        mesh=mesh,
        out_type=jax.ShapeDtypeStruct((NCORES, npad, dt), jnp.float32),
        scratch_types=[
            pltpu.VMEM_SHARED((npad, dt), jnp.float32),
            pltpu.VMEM((SG, CHUNK), jnp.int32),
            pltpu.VMEM((SG, CHUNK), jnp.int32),
            pltpu.VMEM((CHUNK, dt), jnp.float32),
            pltpu.VMEM((CHUNK, dt), jnp.float32),
            pltpu.SemaphoreType.DMA,
            pltpu.SemaphoreType.DMA,
            pltpu.SemaphoreType.DMA,
            pltpu.SemaphoreType.DMA,
        ],
    )
    def sc_scatter(table, gidx, sidx, zeros, out, acc, g_sg, s_sg,
                   rows0, rows1, gsem0, gsem1, ssem0, ssem1):
        rows = (rows0, rows1)
        gsem = (gsem0, gsem1)
        ssem = (ssem0, ssem1)
        c = lax.axis_index("c")
        s = lax.axis_index("s")
        wid = s * NCORES + c
        base = s * rows_per_tile
        pltpu.sync_copy(zeros.at[pl.ds(base, rows_per_tile)],
                        acc.at[pl.ds(base, rows_per_tile)])
        if not gather:
            pltpu.sync_copy(table, rows[0])
        plsc.subcore_barrier()

        def super_group(sg, carry):
            pltpu.sync_copy(sidx.at[wid, pl.ds(sg * SG, SG)], s_sg)
            if gather:
                pltpu.sync_copy(gidx.at[wid, pl.ds(sg * SG, SG)], g_sg)
                gh = [None, None]
                sh = [None, None]
                for k in range(SG):
                    b = k & 1
                    if k >= 2:
                        sh[b].wait()
                    gh[b] = pltpu.async_copy(
                        table.at[g_sg.at[k]], rows[b], gsem[b])
                    if k >= 1:
                        bb = (k - 1) & 1
                        gh[bb].wait()
                        sh[bb] = pltpu.async_copy(
                            rows[bb], acc.at[s_sg.at[k - 1]], ssem[bb],
                            add=True)
                bl = (SG - 1) & 1
                gh[bl].wait()
                sh[bl] = pltpu.async_copy(
                    rows[bl], acc.at[s_sg.at[SG - 1]], ssem[bl], add=True)
                sh[0].wait()
                sh[1].wait()
            else:
                sh = []
                for k in range(SG):
                    sh.append(pltpu.async_copy(
                        rows[0], acc.at[s_sg.at[k]], ssem[k & 1], add=True))
                for h in sh:
                    h.wait()
            return carry

        lax.fori_loop(0, nsg, super_group, 0)
        plsc.subcore_barrier()
        pltpu.sync_copy(acc.at[pl.ds(base, rows_per_tile)],
                        out.at[c, pl.ds(base, rows_per_tile)])

    return sc_scatter


def _dense_layer(x, ax, ar, ac, w, b):
    n, d = x.shape
    blk = 1000
    grid = n // blk

    def body(x_ref, ax_ref, ar_ref, ac_ref, w_ref, b_ref, o_ref):
        a = ax_ref[0] + ax_ref[1]
        r = ar_ref[0] + ar_ref[1]
        cnt = ac_ref[0, :, 0:1] + ac_ref[1, :, 0:1]
        agg = (a + r) / jnp.maximum(cnt, 1.0)
        h = x_ref[...] + agg
        y = jnp.dot(h, w_ref[...], preferred_element_type=jnp.float32)
        o_ref[...] = jnp.maximum(y + b_ref[...], 0.0)

    return pl.pallas_call(
        body,
        grid=(grid,),
        in_specs=[
            pl.BlockSpec((blk, d), lambda i: (i, 0)),
            pl.BlockSpec((NCORES, blk, d), lambda i: (0, i, 0)),
            pl.BlockSpec((NCORES, blk, d), lambda i: (0, i, 0)),
            pl.BlockSpec((NCORES, blk, d), lambda i: (0, i, 0)),
            pl.BlockSpec((d, d), lambda i: (0, 0)),
            pl.BlockSpec((1, d), lambda i: (0, 0)),
        ],
        out_specs=pl.BlockSpec((blk, d), lambda i: (i, 0)),
        out_shape=jax.ShapeDtypeStruct((n, d), jnp.float32),
    )(x, ax, ar, ac, w, b)


def _pool_mlp(x, q, w1, b1, w2, b2):
    n, d = x.shape

    def body(x_ref, q_ref, w1_ref, b1_ref, w2_ref, b2_ref, o_ref):
        g = jnp.mean(x_ref[...], axis=0, keepdims=True)
        comb = jnp.concatenate([q_ref[...], g], axis=1)
        h = jnp.dot(comb, w1_ref[...], preferred_element_type=jnp.float32)
        h = jnp.maximum(h + b1_ref[...], 0.0)
        y = jnp.dot(h, w2_ref[...], preferred_element_type=jnp.float32)
        o_ref[...] = y + b2_ref[...]

    out = pl.pallas_call(
        body,
        out_shape=jax.ShapeDtypeStruct((1, d), jnp.float32),
    )(x, q, w1, b1, w2, b2)
    return out.reshape(d)


def kernel(node_features, edge_index, edge_types, distances, query_idx,
           rel_emb, Wg, bg, W1, b1, W2, b2):
    n, d = node_features.shape
    e = edge_index.shape[1]
    r = rel_emb.shape[0]
    nlayers = Wg.shape[0]

    npad = ((n + 1 + NSUB * 8 - 1) // (NSUB * 8)) * (NSUB * 8)
    per = NW * CHUNK * SG
    e2 = 2 * e
    e2p = ((e2 + per - 1) // per) * per
    nch = e2p // (NW * CHUNK)

    ei = edge_index.astype(jnp.int32)
    src, dst = ei[0], ei[1]
    dist = distances.astype(jnp.int32)
    d_src, d_dst = dist[src], dist[dst]
    mf = d_src == d_dst + 1
    mb = d_dst == d_src + 1

    sidx = jnp.concatenate([jnp.where(mf, dst, n), jnp.where(mb, src, n)])
    gidx = jnp.concatenate([src, dst])
    et = edge_types.astype(jnp.int32)
    tidx = jnp.concatenate([et, et])
    pad = e2p - e2
    sidx = jnp.pad(sidx, (0, pad), constant_values=n).reshape(NW, nch, CHUNK)
    gidx = jnp.pad(gidx, (0, pad)).reshape(NW, nch, CHUNK)
    tidx = jnp.pad(tidx, (0, pad)).reshape(NW, nch, CHUNK)

    zeros_x = jnp.zeros((npad, d), jnp.float32)
    ones_blk = jnp.ones((CHUNK, d), jnp.float32)

    re_pass = _make_sc_scatter(d, npad, nch, True)
    cnt_pass = _make_sc_scatter(d, npad, nch, False)
    x_pass = re_pass

    ar = re_pass(rel_emb, tidx, sidx, zeros_x)
    ac = cnt_pass(ones_blk, tidx, sidx, zeros_x)

    x = node_features
    for l in range(nlayers):
        ax = x_pass(x, gidx, sidx, zeros_x)
        x = _dense_layer(x, ax, ar, ac, Wg[l], bg[l].reshape(1, d))

    q = x[query_idx][None]
    return _pool_mlp(x, q, W1, b1.reshape(1, d), W2, b2.reshape(1, d))
